# Initial kernel scaffold; baseline (speedup 1.0000x reference)
import jax
import jax.numpy as jnp
from jax.experimental import pallas as pl


def kernel(x, expert_weights, W_up, W_gate, W_down, W_pre, W_post, ln_g, ln_b, W_aproj, adapter_W, adapter_ln_g, adapter_ln_b, W_eproj, W_oproj):
    raise NotImplementedError("write your pallas kernel here")


# speedup vs baseline: 3.3626x; 3.3626x over previous
import functools

import jax
import jax.numpy as jnp
from jax import lax
from jax.experimental import pallas as pl

D = 1024
H = 2 * D
A = H // 16
E = 8
B = 2
S = 2048

TA = 512
TB = 512


def _norm(v, eps=1e-5):
    m = jnp.mean(v, axis=-1, keepdims=True)
    var = jnp.mean((v - m) ** 2, axis=-1, keepdims=True)
    return (v - m) * lax.rsqrt(var + eps)


def _dot_t(a, b):
    return lax.dot_general(a, b, (((1,), (1,)), ((), ())),
                           preferred_element_type=jnp.float32)


def _wc_body(we_ref, wo_ref, wc_ref):
    wc_ref[...] = lax.dot_general(we_ref[...], wo_ref[...],
                                  (((0,), (1,)), ((), ())),
                                  preferred_element_type=jnp.float32)


def _stage_a_body(x_ref, wup_ref, wgate_ref, wpre_ref, wpost_ref,
                  g_ref, b_ref,
                  hidden_ref, pre_ref, ain_ref, aout_ref):
    x = x_ref[...]
    up = _dot_t(x, wup_ref[...])
    gate = _dot_t(x, wgate_ref[...])
    hidden = jax.nn.silu(gate) * up
    pre = _dot_t(x, wpre_ref[...])
    g = g_ref[...]
    b = b_ref[...]
    hidden_ref[...] = hidden
    pre_ref[...] = pre
    ain_ref[...] = _norm(pre) * g + b
    aout_ref[...] = _norm(_dot_t(hidden, wpost_ref[...])) * g + b


def _stage_b_body(ew_ref, hidden_ref, pre_ref, ain_t_ref, ain_f_ref,
                  aout_f_ref, waproj_ref, wdown_ref, aw2_ref,
                  alng_ref, alnb_ref, wc_ref, out_ref):
    ain_t = ain_t_ref[...]
    aw = _dot_t(ain_t, aout_f_ref[...])
    aw = jax.nn.silu(jnp.clip(aw, -5.0, 5.0))
    adapt = jnp.dot(aw, ain_f_ref[...],
                    preferred_element_type=jnp.float32)
    hidden = hidden_ref[...] + 0.1 * _dot_t(adapt, waproj_ref[...])
    shared = _dot_t(hidden, wdown_ref[...])

    pre = pre_ref[...]
    eh_all = _dot_t(pre, aw2_ref[...])
    ew = ew_ref[...]
    sel = jnp.zeros_like(pre)
    g_sel = jnp.zeros_like(pre)
    b_sel = jnp.zeros_like(pre)
    for e in range(E):
        m = ew[:, e:e + 1] > 0
        sel = jnp.where(m, eh_all[:, e * A:(e + 1) * A], sel)
        g_sel = jnp.where(m, alng_ref[e:e + 1, :], g_sel)
        b_sel = jnp.where(m, alnb_ref[e:e + 1, :], b_sel)
    any_pos = jnp.max(ew, axis=1, keepdims=True) > 0
    ehn = _norm(sel) * g_sel + b_sel
    eo = jnp.dot(ehn, wc_ref[...], preferred_element_type=jnp.float32)
    out_ref[...] = shared + jnp.where(any_pos, 0.1 * eo, 0.0)


def kernel(x, expert_weights, W_up, W_gate, W_down, W_pre, W_post, ln_g,
           ln_b, W_aproj, adapter_W, adapter_ln_g, adapter_ln_b, W_eproj,
           W_oproj):
    BS = B * S
    x2 = x.reshape(BS, D)
    ew2 = expert_weights.reshape(BS, E)
    aw2 = adapter_W.reshape(E * A, A)
    g2 = ln_g.reshape(1, A)
    b2 = ln_b.reshape(1, A)

    wc = pl.pallas_call(
        _wc_body,
        out_shape=jax.ShapeDtypeStruct((A, D), jnp.float32),
    )(W_eproj, W_oproj)

    full = lambda shape: pl.BlockSpec(shape, lambda i: (0,) * len(shape))
    hidden, pre, ain, aout = pl.pallas_call(
        _stage_a_body,
        grid=(BS // TA,),
        in_specs=[
            pl.BlockSpec((TA, D), lambda i: (i, 0)),
            full((H, D)), full((H, D)), full((A, D)), full((A, H)),
            full((1, A)), full((1, A)),
        ],
        out_specs=[
            pl.BlockSpec((TA, H), lambda i: (i, 0)),
            pl.BlockSpec((TA, A), lambda i: (i, 0)),
            pl.BlockSpec((TA, A), lambda i: (i, 0)),
            pl.BlockSpec((TA, A), lambda i: (i, 0)),
        ],
        out_shape=[
            jax.ShapeDtypeStruct((BS, H), jnp.float32),
            jax.ShapeDtypeStruct((BS, A), jnp.float32),
            jax.ShapeDtypeStruct((BS, A), jnp.float32),
            jax.ShapeDtypeStruct((BS, A), jnp.float32),
        ],
    )(x2, W_up, W_gate, W_pre, W_post, g2, b2)

    nt = S // TB
    tile = lambda w: pl.BlockSpec((TB, w), lambda bi, ti: (bi * nt + ti, 0))
    fullb = lambda shape: pl.BlockSpec(shape, lambda bi, ti: (0,) * len(shape))
    out = pl.pallas_call(
        _stage_b_body,
        grid=(B, nt),
        in_specs=[
            tile(E),
            tile(H),
            tile(A),
            tile(A),
            pl.BlockSpec((S, A), lambda bi, ti: (bi, 0)),
            pl.BlockSpec((S, A), lambda bi, ti: (bi, 0)),
            fullb((H, A)),
            fullb((D, H)),
            fullb((E * A, A)),
            fullb((E, A)),
            fullb((E, A)),
            fullb((A, D)),
        ],
        out_specs=tile(D),
        out_shape=jax.ShapeDtypeStruct((BS, D), jnp.float32),
    )(ew2, hidden, pre, ain, ain, aout, W_aproj, W_down, aw2,
      adapter_ln_g, adapter_ln_b, wc)
    return out.reshape(B, S, D)
